# baseline (speedup 1.0000x reference)
import functools

import jax
import jax.numpy as jnp
import numpy as np
from jax.experimental import pallas as pl
from jax.experimental.pallas import tpu as pltpu

EPS = 1e-10
K_DIM = 32
TOPK = 32

_PAIRS = [(i, j) for i in range(TOPK) for j in range(TOPK)
          if (i + 1) * (j + 1) <= TOPK]
_NCAND = 128


def _combine_consts():
    a = np.zeros((TOPK, _NCAND), np.float32)
    b = np.zeros((TOPK, _NCAND), np.float32)
    cb = np.zeros((2, _NCAND), np.float32)
    cb[0, :] = -1000.0
    cb[1, :] = -1.0
    for t, (i, j) in enumerate(_PAIRS):
        a[i, t] = 1.0
        b[j, t] = 1.0
        cb[0, t] = 0.0
        cb[1, t] = float(1023 - (i * TOPK + j))
    return a, b, cb

_dotf = functools.partial(
    jnp.dot,
    preferred_element_type=jnp.float32,
    precision=jax.lax.Precision.HIGHEST,
)
_dotd = functools.partial(jnp.dot, preferred_element_type=jnp.float32)


def _pkm_kernel(x_ref, wt_ref, b_ref, k0_ref, k1_ref, a_ref, bm_ref, cb_ref,
                scores_ref, idx_ref):
    R = x_ref.shape[0]
    NK = k0_ref.shape[1]

    q = _dotd(x_ref[...], wt_ref[...])
    q = jnp.maximum(q + b_ref[...], 0.0)
    q0 = q[:, :K_DIM]
    q1 = q[:, K_DIM:]
    q0 = q0 / (jnp.sqrt(jnp.sum(q0 * q0, axis=1, keepdims=True)) + EPS)
    q1 = q1 / (jnp.sqrt(jnp.sum(q1 * q1, axis=1, keepdims=True)) + EPS)

    k0 = k0_ref[...]
    k1 = k1_ref[...]
    kn0 = jnp.sqrt(jnp.sum(k0 * k0, axis=0, keepdims=True)) + EPS
    kn1 = jnp.sqrt(jnp.sum(k1 * k1, axis=0, keepdims=True)) + EPS
    l0 = _dotd(q0, k0) / kn0
    l1 = _dotd(q1, k1) / kn1
    logits = jnp.concatenate([l0, l1], axis=0)

    col32 = jax.lax.broadcasted_iota(jnp.int32, (1, TOPK), 1)

    NSEG = 16
    SEGW = NK // NSEG
    lgs = logits.reshape(2 * R, NSEG, SEGW)
    revio3 = (4095 - (jax.lax.broadcasted_iota(
        jnp.int32, (1, NSEG, SEGW), 1) * SEGW + jax.lax.broadcasted_iota(
        jnp.int32, (1, NSEG, SEGW), 2))).astype(jnp.float32)
    NCW = TOPK * NSEG
    ii8 = jax.lax.broadcasted_iota(jnp.int32, (NSEG, NCW), 0)
    cc8 = jax.lax.broadcasted_iota(jnp.int32, (NSEG, NCW), 1)
    iocw = jax.lax.broadcasted_iota(jnp.int32, (1, NCW), 1)

    def seg_cond(state):
        _, _, _, t, cert = state
        return jnp.logical_and(t < TOPK, jnp.logical_not(cert))

    def seg_body(state):
        lgs, cand, icand, t, _ = state
        m_seg = jnp.max(lgs, axis=2, keepdims=True)
        eq = lgs == m_seg
        wr = jnp.where(eq, revio3, -1.0)
        cr = jnp.max(wr, axis=2, keepdims=True)
        sel = jnp.logical_and(eq, wr == cr)
        lgs = jnp.where(sel, -2.0, lgs)
        et = (cc8 == ii8 + NSEG * t).astype(jnp.float32)
        cand = cand + _dotf(m_seg[:, :, 0], et)
        icand = icand + _dotf(4095.0 - cr[:, :, 0], et)
        theta = jnp.max(m_seg[:, :, 0], axis=1, keepdims=True)
        written = iocw < NSEG * (t + 1)
        cnt = jnp.sum(jnp.where(
            jnp.logical_and(written, cand > theta), 1.0, 0.0), axis=1)
        cert = jnp.all(cnt >= float(TOPK))
        return lgs, cand, icand, t + 1, cert

    cand0 = jnp.zeros((2 * R, NCW), jnp.float32)
    _, cand, icand, t_end, _ = jax.lax.while_loop(
        seg_cond, seg_body,
        (lgs, cand0, cand0, jnp.int32(0), jnp.bool_(False)))
    cand = jnp.where(iocw >= NSEG * t_end, -2.0, cand)

    revi2 = 4095.0 - icand

    def merge_body(k, carry):
        cand, s01, i01 = carry
        m = jnp.max(cand, axis=1, keepdims=True)
        eq = cand == m
        wr = jnp.where(eq, revi2, -1.0)
        cr = jnp.max(wr, axis=1, keepdims=True)
        sel = jnp.logical_and(eq, wr == cr)
        cand = jnp.where(sel, -2.0, cand)
        onek = col32 == k
        s01 = jnp.where(onek, m, s01)
        i01 = jnp.where(onek, 4095.0 - cr, i01)
        return cand, s01, i01

    zeros = jnp.zeros((2 * R, TOPK), jnp.float32)
    _, s01, i01 = jax.lax.fori_loop(
        0, TOPK, merge_body, (cand, zeros, zeros))
    s0, s1 = s01[:R], s01[R:]
    i0, i1 = i01[:R], i01[R:]

    ea = a_ref[...]
    eb = bm_ref[...]
    sums = _dotf(s0, ea) + _dotf(s1, eb) + cb_ref[0:1, :]
    all_idx = _dotf(i0, ea) * 4096.0 + _dotf(i1, eb)

    revio_c = jnp.broadcast_to(cb_ref[1:2, :], (R, _NCAND))

    def comb_body(k, carry):
        sm, best, fi = carry
        m2 = jnp.max(sm, axis=1, keepdims=True)
        eq2 = sm == m2
        cr = jnp.max(jnp.where(eq2, revio_c, -1.0), axis=1, keepdims=True)
        sel = jnp.logical_and(eq2, revio_c == cr)
        fik = jnp.max(jnp.where(sel, all_idx, -1.0), axis=1, keepdims=True)
        sm = jnp.where(sel, -5.0, sm)
        onek = col32 == k
        best = jnp.where(onek, m2, best)
        fi = jnp.where(onek, fik, fi)
        return sm, best, fi

    zr = jnp.zeros((R, TOPK), jnp.float32)
    _, best, fi = jax.lax.fori_loop(0, TOPK, comb_body, (sums, zr, zr))

    e = jnp.exp(best - jnp.max(best, axis=1, keepdims=True))
    scores_ref[...] = e / jnp.sum(e, axis=1, keepdims=True)
    idx_ref[...] = fi.astype(jnp.int32)


def _run(x, wt, b2, k0, k1, a128, b128, cb, block_rows, interpret=False):
    bs, in_dim = x.shape
    nk = k0.shape[1]
    grid = (bs // block_rows,)
    return pl.pallas_call(
        _pkm_kernel,
        grid=grid,
        in_specs=[
            pl.BlockSpec((block_rows, in_dim), lambda i: (i, 0)),
            pl.BlockSpec((in_dim, 2 * K_DIM), lambda i: (0, 0)),
            pl.BlockSpec((1, 2 * K_DIM), lambda i: (0, 0)),
            pl.BlockSpec((K_DIM, nk), lambda i: (0, 0)),
            pl.BlockSpec((K_DIM, nk), lambda i: (0, 0)),
            pl.BlockSpec((TOPK, _NCAND), lambda i: (0, 0)),
            pl.BlockSpec((TOPK, _NCAND), lambda i: (0, 0)),
            pl.BlockSpec((2, _NCAND), lambda i: (0, 0)),
        ],
        out_specs=[
            pl.BlockSpec((block_rows, TOPK), lambda i: (i, 0)),
            pl.BlockSpec((block_rows, TOPK), lambda i: (i, 0)),
        ],
        out_shape=[
            jax.ShapeDtypeStruct((bs, TOPK), jnp.float32),
            jax.ShapeDtypeStruct((bs, TOPK), jnp.int32),
        ],
        compiler_params=pltpu.CompilerParams(
            dimension_semantics=("arbitrary",),
        ),
        interpret=interpret,
    )(x, wt, b2, k0, k1, a128, b128, cb)


def kernel(x, W, b, keys, block_rows=128, interpret=False):
    wt = W.T
    b2 = b.reshape(1, -1)
    kt = jnp.transpose(keys, (1, 2, 0))
    a128, b128, cb = _combine_consts()
    scores, idx = _run(x, wt, b2, kt[0], kt[1],
                       jnp.asarray(a128), jnp.asarray(b128), jnp.asarray(cb),
                       block_rows, interpret=interpret)
    return scores, idx


# speedup vs baseline: 1.1219x; 1.1219x over previous
import functools

import jax
import jax.numpy as jnp
import numpy as np
from jax.experimental import pallas as pl
from jax.experimental.pallas import tpu as pltpu

EPS = 1e-10
K_DIM = 32
TOPK = 32

_PAIRS = [(i, j) for i in range(TOPK) for j in range(TOPK)
          if (i + 1) * (j + 1) <= TOPK]
_NCAND = 128


def _combine_consts():
    a = np.zeros((TOPK, _NCAND), np.float32)
    b = np.zeros((TOPK, _NCAND), np.float32)
    cb = np.zeros((2, _NCAND), np.float32)
    cb[0, :] = -1000.0
    cb[1, :] = -1.0
    for t, (i, j) in enumerate(_PAIRS):
        a[i, t] = 1.0
        b[j, t] = 1.0
        cb[0, t] = 0.0
        cb[1, t] = float(1023 - (i * TOPK + j))
    return a, b, cb

_dotf = functools.partial(
    jnp.dot,
    preferred_element_type=jnp.float32,
    precision=jax.lax.Precision.HIGHEST,
)
_dotd = functools.partial(jnp.dot, preferred_element_type=jnp.float32)


def _pkm_kernel(x_ref, wt_ref, b_ref, k0_ref, k1_ref, a_ref, bm_ref, cb_ref,
                scores_ref, idx_ref):
    R = x_ref.shape[0]
    NK = k0_ref.shape[1]

    q = _dotd(x_ref[...], wt_ref[...])
    q = jnp.maximum(q + b_ref[...], 0.0)
    q0 = q[:, :K_DIM]
    q1 = q[:, K_DIM:]
    q0 = q0 / (jnp.sqrt(jnp.sum(q0 * q0, axis=1, keepdims=True)) + EPS)
    q1 = q1 / (jnp.sqrt(jnp.sum(q1 * q1, axis=1, keepdims=True)) + EPS)

    k0 = k0_ref[...]
    k1 = k1_ref[...]
    kn0 = jnp.sqrt(jnp.sum(k0 * k0, axis=0, keepdims=True)) + EPS
    kn1 = jnp.sqrt(jnp.sum(k1 * k1, axis=0, keepdims=True)) + EPS
    l0 = _dotd(q0, k0) / kn0
    l1 = _dotd(q1, k1) / kn1
    logits = jnp.concatenate([l0, l1], axis=0)

    col32 = jax.lax.broadcasted_iota(jnp.int32, (1, TOPK), 1)

    NSEG = 8
    SEGW = NK // NSEG
    lgs = logits.reshape(2 * R, NSEG, SEGW)
    revio3 = (4095 - (jax.lax.broadcasted_iota(
        jnp.int32, (1, NSEG, SEGW), 1) * SEGW + jax.lax.broadcasted_iota(
        jnp.int32, (1, NSEG, SEGW), 2))).astype(jnp.float32)
    NCW = TOPK * NSEG
    ii8 = jax.lax.broadcasted_iota(jnp.int32, (NSEG, NCW), 0)
    cc8 = jax.lax.broadcasted_iota(jnp.int32, (NSEG, NCW), 1)
    iocw = jax.lax.broadcasted_iota(jnp.int32, (1, NCW), 1)

    def seg_cond(state):
        _, _, _, t, cert = state
        return jnp.logical_and(t < TOPK, jnp.logical_not(cert))

    def seg_body(state):
        lgs, cand, icand, t, _ = state
        m_seg = jnp.max(lgs, axis=2, keepdims=True)
        eq = lgs == m_seg
        wr = jnp.where(eq, revio3, -1.0)
        cr = jnp.max(wr, axis=2, keepdims=True)
        sel = jnp.logical_and(eq, wr == cr)
        lgs = jnp.where(sel, -2.0, lgs)
        et = (cc8 == ii8 + NSEG * t).astype(jnp.float32)
        cand = cand + _dotf(m_seg[:, :, 0], et)
        icand = icand + _dotf(4095.0 - cr[:, :, 0], et)
        theta = jnp.max(m_seg[:, :, 0], axis=1, keepdims=True)
        written = iocw < NSEG * (t + 1)
        cnt = jnp.sum(jnp.where(
            jnp.logical_and(written, cand > theta), 1.0, 0.0), axis=1)
        cert = jnp.all(cnt >= float(TOPK))
        return lgs, cand, icand, t + 1, cert

    cand0 = jnp.zeros((2 * R, NCW), jnp.float32)
    _, cand, icand, t_end, _ = jax.lax.while_loop(
        seg_cond, seg_body,
        (lgs, cand0, cand0, jnp.int32(0), jnp.bool_(False)))
    cand = jnp.where(iocw >= NSEG * t_end, -2.0, cand)

    revi2 = 4095.0 - icand

    def merge_body(k, carry):
        cand, s01, i01 = carry
        m = jnp.max(cand, axis=1, keepdims=True)
        eq = cand == m
        wr = jnp.where(eq, revi2, -1.0)
        cr = jnp.max(wr, axis=1, keepdims=True)
        sel = jnp.logical_and(eq, wr == cr)
        cand = jnp.where(sel, -2.0, cand)
        onek = col32 == k
        s01 = jnp.where(onek, m, s01)
        i01 = jnp.where(onek, 4095.0 - cr, i01)
        return cand, s01, i01

    zeros = jnp.zeros((2 * R, TOPK), jnp.float32)
    _, s01, i01 = jax.lax.fori_loop(
        0, TOPK, merge_body, (cand, zeros, zeros))
    s0, s1 = s01[:R], s01[R:]
    i0, i1 = i01[:R], i01[R:]

    ea = a_ref[...]
    eb = bm_ref[...]
    sums = _dotf(s0, ea) + _dotf(s1, eb) + cb_ref[0:1, :]
    all_idx = _dotf(i0, ea) * 4096.0 + _dotf(i1, eb)

    revio_c = jnp.broadcast_to(cb_ref[1:2, :], (R, _NCAND))

    def comb_body(k, carry):
        sm, best, fi = carry
        m2 = jnp.max(sm, axis=1, keepdims=True)
        eq2 = sm == m2
        cr = jnp.max(jnp.where(eq2, revio_c, -1.0), axis=1, keepdims=True)
        sel = jnp.logical_and(eq2, revio_c == cr)
        fik = jnp.max(jnp.where(sel, all_idx, -1.0), axis=1, keepdims=True)
        sm = jnp.where(sel, -5.0, sm)
        onek = col32 == k
        best = jnp.where(onek, m2, best)
        fi = jnp.where(onek, fik, fi)
        return sm, best, fi

    zr = jnp.zeros((R, TOPK), jnp.float32)
    _, best, fi = jax.lax.fori_loop(0, TOPK, comb_body, (sums, zr, zr))

    e = jnp.exp(best - jnp.max(best, axis=1, keepdims=True))
    scores_ref[...] = e / jnp.sum(e, axis=1, keepdims=True)
    idx_ref[...] = fi.astype(jnp.int32)


def _run(x, wt, b2, k0, k1, a128, b128, cb, block_rows, interpret=False):
    bs, in_dim = x.shape
    nk = k0.shape[1]
    grid = (bs // block_rows,)
    return pl.pallas_call(
        _pkm_kernel,
        grid=grid,
        in_specs=[
            pl.BlockSpec((block_rows, in_dim), lambda i: (i, 0)),
            pl.BlockSpec((in_dim, 2 * K_DIM), lambda i: (0, 0)),
            pl.BlockSpec((1, 2 * K_DIM), lambda i: (0, 0)),
            pl.BlockSpec((K_DIM, nk), lambda i: (0, 0)),
            pl.BlockSpec((K_DIM, nk), lambda i: (0, 0)),
            pl.BlockSpec((TOPK, _NCAND), lambda i: (0, 0)),
            pl.BlockSpec((TOPK, _NCAND), lambda i: (0, 0)),
            pl.BlockSpec((2, _NCAND), lambda i: (0, 0)),
        ],
        out_specs=[
            pl.BlockSpec((block_rows, TOPK), lambda i: (i, 0)),
            pl.BlockSpec((block_rows, TOPK), lambda i: (i, 0)),
        ],
        out_shape=[
            jax.ShapeDtypeStruct((bs, TOPK), jnp.float32),
            jax.ShapeDtypeStruct((bs, TOPK), jnp.int32),
        ],
        compiler_params=pltpu.CompilerParams(
            dimension_semantics=("arbitrary",),
        ),
        interpret=interpret,
    )(x, wt, b2, k0, k1, a128, b128, cb)


def kernel(x, W, b, keys, block_rows=256, interpret=False):
    wt = W.T
    b2 = b.reshape(1, -1)
    kt = jnp.transpose(keys, (1, 2, 0))
    a128, b128, cb = _combine_consts()
    scores, idx = _run(x, wt, b2, kt[0], kt[1],
                       jnp.asarray(a128), jnp.asarray(b128), jnp.asarray(cb),
                       block_rows, interpret=interpret)
    return scores, idx


# speedup vs baseline: 1.1508x; 1.0258x over previous
import functools

import jax
import jax.numpy as jnp
import numpy as np
from jax.experimental import pallas as pl
from jax.experimental.pallas import tpu as pltpu

EPS = 1e-10
K_DIM = 32
TOPK = 32

_PAIRS = [(i, j) for i in range(TOPK) for j in range(TOPK)
          if (i + 1) * (j + 1) <= TOPK]
_NCAND = 128


def _combine_consts():
    a = np.zeros((TOPK, _NCAND), np.float32)
    b = np.zeros((TOPK, _NCAND), np.float32)
    cb = np.zeros((2, _NCAND), np.float32)
    cb[0, :] = -1000.0
    cb[1, :] = -1.0
    for t, (i, j) in enumerate(_PAIRS):
        a[i, t] = 1.0
        b[j, t] = 1.0
        cb[0, t] = 0.0
        cb[1, t] = float(1023 - (i * TOPK + j))
    return a, b, cb

_dotf = functools.partial(
    jnp.dot,
    preferred_element_type=jnp.float32,
    precision=jax.lax.Precision.HIGHEST,
)
_dotd = functools.partial(jnp.dot, preferred_element_type=jnp.float32)


def _pkm_kernel(x_ref, wt_ref, b_ref, k0_ref, k1_ref, a_ref, bm_ref, cb_ref,
                scores_ref, idx_ref):
    R = x_ref.shape[0]
    NK = k0_ref.shape[1]

    q = _dotd(x_ref[...], wt_ref[...])
    q = jnp.maximum(q + b_ref[...], 0.0)
    q0 = q[:, :K_DIM]
    q1 = q[:, K_DIM:]
    q0 = q0 / (jnp.sqrt(jnp.sum(q0 * q0, axis=1, keepdims=True)) + EPS)
    q1 = q1 / (jnp.sqrt(jnp.sum(q1 * q1, axis=1, keepdims=True)) + EPS)

    k0 = k0_ref[...]
    k1 = k1_ref[...]
    kn0 = jnp.sqrt(jnp.sum(k0 * k0, axis=0, keepdims=True)) + EPS
    kn1 = jnp.sqrt(jnp.sum(k1 * k1, axis=0, keepdims=True)) + EPS
    l0 = _dotd(q0, k0) / kn0
    l1 = _dotd(q1, k1) / kn1
    logits = jnp.concatenate([l0, l1], axis=0)

    col32 = jax.lax.broadcasted_iota(jnp.int32, (1, TOPK), 1)

    NSEG = 8
    SEGW = NK // NSEG
    lgs = logits.reshape(2 * R, NSEG, SEGW)
    revio3 = (4095 - (jax.lax.broadcasted_iota(
        jnp.int32, (1, NSEG, SEGW), 1) * SEGW + jax.lax.broadcasted_iota(
        jnp.int32, (1, NSEG, SEGW), 2))).astype(jnp.float32)
    NCW = TOPK * NSEG
    ii8 = jax.lax.broadcasted_iota(jnp.int32, (NSEG, NCW), 0)
    cc8 = jax.lax.broadcasted_iota(jnp.int32, (NSEG, NCW), 1)
    iocw = jax.lax.broadcasted_iota(jnp.int32, (1, NCW), 1)

    def seg_cond(state):
        _, _, _, t, cert = state
        return jnp.logical_and(t < TOPK, jnp.logical_not(cert))

    def seg_body(state):
        lgs, cand, icand, t, _ = state
        m_seg = jnp.max(lgs, axis=2, keepdims=True)
        eq = lgs == m_seg
        wr = jnp.where(eq, revio3, -1.0)
        cr = jnp.max(wr, axis=2, keepdims=True)
        sel = jnp.logical_and(eq, wr == cr)
        lgs = jnp.where(sel, -2.0, lgs)
        et = (cc8 == ii8 + NSEG * t).astype(jnp.float32)
        cand = cand + _dotf(m_seg[:, :, 0], et)
        icand = icand + _dotf(4095.0 - cr[:, :, 0], et)
        theta = jnp.max(m_seg[:, :, 0], axis=1, keepdims=True)
        written = iocw < NSEG * (t + 1)
        cnt = jnp.sum(jnp.where(
            jnp.logical_and(written, cand > theta), 1.0, 0.0), axis=1)
        cert = jnp.all(cnt >= float(TOPK))
        return lgs, cand, icand, t + 1, cert

    cand0 = jnp.zeros((2 * R, NCW), jnp.float32)
    _, cand, icand, t_end, _ = jax.lax.while_loop(
        seg_cond, seg_body,
        (lgs, cand0, cand0, jnp.int32(0), jnp.bool_(False)))
    cand = jnp.where(iocw >= NSEG * t_end, -2.0, cand)

    revi2 = 4095.0 - icand

    def merge_body(k, carry):
        cand, s01, i01 = carry
        m = jnp.max(cand, axis=1, keepdims=True)
        eq = cand == m
        wr = jnp.where(eq, revi2, -1.0)
        cr = jnp.max(wr, axis=1, keepdims=True)
        sel = jnp.logical_and(eq, wr == cr)
        cand = jnp.where(sel, -2.0, cand)
        onek = col32 == k
        s01 = jnp.where(onek, m, s01)
        i01 = jnp.where(onek, 4095.0 - cr, i01)
        return cand, s01, i01

    zeros = jnp.zeros((2 * R, TOPK), jnp.float32)
    _, s01, i01 = jax.lax.fori_loop(
        0, TOPK, merge_body, (cand, zeros, zeros))
    s0, s1 = s01[:R], s01[R:]
    i0, i1 = i01[:R], i01[R:]

    ea = a_ref[...]
    eb = bm_ref[...]
    sums = _dotf(s0, ea) + _dotf(s1, eb) + cb_ref[0:1, :]
    all_idx = _dotf(i0, ea) * 4096.0 + _dotf(i1, eb)

    revio_c = jnp.broadcast_to(cb_ref[1:2, :], (R, _NCAND))

    def comb_body(k, carry):
        sm, best, fi = carry
        m2 = jnp.max(sm, axis=1, keepdims=True)
        eq2 = sm == m2
        cr = jnp.max(jnp.where(eq2, revio_c, -1.0), axis=1, keepdims=True)
        sel = jnp.logical_and(eq2, revio_c == cr)
        fik = jnp.max(jnp.where(sel, all_idx, -1.0), axis=1, keepdims=True)
        sm = jnp.where(sel, -5.0, sm)
        onek = col32 == k
        best = jnp.where(onek, m2, best)
        fi = jnp.where(onek, fik, fi)
        return sm, best, fi

    zr = jnp.zeros((R, TOPK), jnp.float32)
    _, best, fi = jax.lax.fori_loop(0, TOPK, comb_body, (sums, zr, zr))

    e = jnp.exp(best - jnp.max(best, axis=1, keepdims=True))
    scores_ref[...] = e / jnp.sum(e, axis=1, keepdims=True)
    idx_ref[...] = fi.astype(jnp.int32)


def _run(x, wt, b2, k0, k1, a128, b128, cb, block_rows, interpret=False):
    bs, in_dim = x.shape
    nk = k0.shape[1]
    grid = (bs // block_rows,)
    return pl.pallas_call(
        _pkm_kernel,
        grid=grid,
        in_specs=[
            pl.BlockSpec((block_rows, in_dim), lambda i: (i, 0)),
            pl.BlockSpec((in_dim, 2 * K_DIM), lambda i: (0, 0)),
            pl.BlockSpec((1, 2 * K_DIM), lambda i: (0, 0)),
            pl.BlockSpec((K_DIM, nk), lambda i: (0, 0)),
            pl.BlockSpec((K_DIM, nk), lambda i: (0, 0)),
            pl.BlockSpec((TOPK, _NCAND), lambda i: (0, 0)),
            pl.BlockSpec((TOPK, _NCAND), lambda i: (0, 0)),
            pl.BlockSpec((2, _NCAND), lambda i: (0, 0)),
        ],
        out_specs=[
            pl.BlockSpec((block_rows, TOPK), lambda i: (i, 0)),
            pl.BlockSpec((block_rows, TOPK), lambda i: (i, 0)),
        ],
        out_shape=[
            jax.ShapeDtypeStruct((bs, TOPK), jnp.float32),
            jax.ShapeDtypeStruct((bs, TOPK), jnp.int32),
        ],
        compiler_params=pltpu.CompilerParams(
            dimension_semantics=("arbitrary",),
        ),
        interpret=interpret,
    )(x, wt, b2, k0, k1, a128, b128, cb)


def kernel(x, W, b, keys, block_rows=512, interpret=False):
    wt = W.T
    b2 = b.reshape(1, -1)
    kt = jnp.transpose(keys, (1, 2, 0))
    a128, b128, cb = _combine_consts()
    scores, idx = _run(x, wt, b2, kt[0], kt[1],
                       jnp.asarray(a128), jnp.asarray(b128), jnp.asarray(cb),
                       block_rows, interpret=interpret)
    return scores, idx
